# baseline (speedup 1.0000x reference)
import functools

import jax
import jax.numpy as jnp
from jax.experimental import pallas as pl
from jax.experimental.pallas import tpu as pltpu


def _vb_kernel(Hf, Wf, CK, bbox_ref, obj_ref, frame_ref, wobj_ref, bobj_ref,
               wkey_ref, bkey_ref, wval_ref, bval_ref, wctx_ref, bctx_ref,
               out_ref, kv_acc_ref, q_acc_ref, att_acc_ref, wctx_scr_ref,
               mask_scr_ref):
    j = pl.program_id(0)
    b = pl.program_id(1)
    nj = pl.num_programs(0)
    nb = pl.num_programs(1)
    s = j * nb + b
    f32 = jnp.float32
    bf16 = jnp.bfloat16
    N = bbox_ref.shape[0]
    P = frame_ref.shape[3]
    Dq = wobj_ref.shape[1]

    @pl.when(s == 0)
    def _():
        bx = bbox_ref[...]
        x1 = jnp.minimum(bx[:, 0:1], bx[:, 2:3])
        x2 = jnp.maximum(bx[:, 0:1], bx[:, 2:3])
        y1 = jnp.minimum(bx[:, 1:2], bx[:, 3:4])
        y2 = jnp.maximum(bx[:, 1:2], bx[:, 3:4])
        pos = jax.lax.broadcasted_iota(jnp.int32, (N, P), 1)
        yc = ((pos // Wf).astype(f32) + 0.5) * (1.0 / Hf)
        xc = ((pos % Wf).astype(f32) + 0.5) * (1.0 / Wf)
        mask = ((yc >= y1) & (yc <= y2) & (xc >= x1) & (xc <= x2)).astype(f32)
        denom = jnp.maximum(jnp.sum(mask, axis=1, keepdims=True), 1.0)
        mask_scr_ref[:, :P] = mask
        mask_scr_ref[:, P:P + 1] = 1.0 / denom

    wctx_scr_ref[:, pl.ds(s * wctx_ref.shape[1], wctx_ref.shape[1])] = (
        wctx_ref[...])

    onehot = (obj_ref[...] == b).astype(f32)
    mb = (mask_scr_ref[:, :P] * onehot).astype(bf16)
    inv_denom = mask_scr_ref[:, P:P + 1]

    frame_c = frame_ref[0, 0].astype(bf16)

    pooledc = jax.lax.dot_general(mb, frame_c, (((1,), (1,)), ((), ())),
                                  preferred_element_type=f32) * inv_denom
    out_col = j * CK

    @pl.when(b == 0)
    def _():
        out_ref[:, pl.ds(out_col, CK)] = pooledc

    @pl.when(b != 0)
    def _():
        out_ref[:, pl.ds(out_col, CK)] += pooledc

    q_part = jnp.dot(pooledc.astype(bf16), wobj_ref[...].astype(bf16),
                     preferred_element_type=f32)
    kv_part = jnp.concatenate([
        jax.lax.dot_general(frame_c, wkey_ref[...].astype(bf16),
                            (((0,), (0,)), ((), ())),
                            preferred_element_type=f32),
        jax.lax.dot_general(frame_c, wval_ref[...].astype(bf16),
                            (((0,), (0,)), ((), ())),
                            preferred_element_type=f32)], axis=1)

    @pl.when(s == 0)
    def _():
        q_acc_ref[...] = q_part

    @pl.when(s != 0)
    def _():
        q_acc_ref[...] += q_part

    @pl.when(j == 0)
    def _():
        kv_acc_ref[b] = kv_part

    @pl.when(j != 0)
    def _():
        kv_acc_ref[b] += kv_part

    @pl.when(j == nj - 1)
    def _():
        q = jnp.maximum(q_acc_ref[...] + bobj_ref[...], 0.0)
        kv = kv_acc_ref[b]
        keym = jnp.maximum(kv[:, :Dq] + bkey_ref[...], 0.0)
        valm = jnp.maximum(kv[:, Dq:] + bval_ref[...], 0.0)
        scores = jax.lax.dot_general(q, keym, (((1,), (1,)), ((), ())),
                                     preferred_element_type=f32)
        m = jnp.max(scores, axis=1, keepdims=True)
        e = jnp.exp(scores - m)
        attn = (e / jnp.sum(e, axis=1, keepdims=True)) * onehot
        att = jnp.dot(attn, valm, preferred_element_type=f32)

        @pl.when(b == 0)
        def _():
            att_acc_ref[...] = att

        @pl.when(b != 0)
        def _():
            att_acc_ref[...] += att

        @pl.when(b == nb - 1)
        def _():
            C = nj * CK
            ctx = jnp.maximum(
                jnp.dot(att_acc_ref[...].astype(bf16),
                        wctx_scr_ref[...].astype(bf16),
                        preferred_element_type=f32) + bctx_ref[...], 0.0)
            out_ref[:, C:] = ctx


@jax.jit
def kernel(frame_deep_features, bboxes, obj_slicing, W_obj, b_obj, W_key,
           b_key, W_val, b_val, W_ctx, b_ctx):
    B, C, Hf, Wf = frame_deep_features.shape
    N = bboxes.shape[0]
    P = Hf * Wf
    Dq = W_obj.shape[1]
    Dc = W_ctx.shape[1]
    NJ = 2
    CK = C // NJ
    DCK = Dc // (NJ * B)
    frame_flat = frame_deep_features.reshape(B, NJ, CK, P)
    obj2 = obj_slicing.reshape(N, 1)

    return pl.pallas_call(
        functools.partial(_vb_kernel, Hf, Wf, CK),
        grid=(NJ, B),
        in_specs=[
            pl.BlockSpec((N, 4), lambda j, b: (0, 0)),
            pl.BlockSpec((N, 1), lambda j, b: (0, 0)),
            pl.BlockSpec((1, 1, CK, P), lambda j, b: (b, j, 0, 0)),
            pl.BlockSpec((CK, Dq), lambda j, b: (j, 0)),
            pl.BlockSpec((1, Dq), lambda j, b: (0, 0)),
            pl.BlockSpec((CK, Dq), lambda j, b: (j, 0)),
            pl.BlockSpec((1, Dq), lambda j, b: (0, 0)),
            pl.BlockSpec((CK, Dq), lambda j, b: (j, 0)),
            pl.BlockSpec((1, Dq), lambda j, b: (0, 0)),
            pl.BlockSpec((Dq, DCK), lambda j, b: (0, j * B + b)),
            pl.BlockSpec((1, Dc), lambda j, b: (0, 0)),
        ],
        out_specs=pl.BlockSpec((N, C + Dc), lambda j, b: (0, 0)),
        out_shape=jax.ShapeDtypeStruct((N, C + Dc), jnp.float32),
        scratch_shapes=[
            pltpu.VMEM((B, P, 2 * Dq), jnp.float32),
            pltpu.VMEM((N, Dq), jnp.float32),
            pltpu.VMEM((N, Dq), jnp.float32),
            pltpu.VMEM((Dq, Dc), jnp.float32),
            pltpu.VMEM((N, P + 1), jnp.float32),
        ],
    )(bboxes, obj2, frame_flat, W_obj, b_obj.reshape(1, Dq),
      W_key, b_key.reshape(1, Dq), W_val,
      b_val.reshape(1, Dq), W_ctx, b_ctx.reshape(1, Dc))


# speedup vs baseline: 1.6078x; 1.6078x over previous
import functools

import jax
import jax.numpy as jnp
from jax.experimental import pallas as pl
from jax.experimental.pallas import tpu as pltpu


def _vb_kernel(Hf, Wf, bbox_ref, obj_ref, frame_ref, wobj_ref, bobj_ref,
               wkey_ref, bkey_ref, wval_ref, bval_ref, wctx_ref, bctx_ref,
               out_ref, att_acc_ref, wobj_bf_ref, wkey_bf_ref, wval_bf_ref,
               wctx_bf_ref):
    b = pl.program_id(0)
    nb = pl.num_programs(0)
    f32 = jnp.float32
    bf16 = jnp.bfloat16
    N = bbox_ref.shape[0]
    C, P = frame_ref.shape[1], frame_ref.shape[2]

    @pl.when(b == 0)
    def _():
        wobj_bf_ref[...] = wobj_ref[...].astype(bf16)
        wkey_bf_ref[...] = wkey_ref[...].astype(bf16)
        wval_bf_ref[...] = wval_ref[...].astype(bf16)
        wctx_bf_ref[...] = wctx_ref[...].astype(bf16)

    bx = bbox_ref[...]
    x1 = jnp.minimum(bx[:, 0:1], bx[:, 2:3])
    x2 = jnp.maximum(bx[:, 0:1], bx[:, 2:3])
    y1 = jnp.minimum(bx[:, 1:2], bx[:, 3:4])
    y2 = jnp.maximum(bx[:, 1:2], bx[:, 3:4])
    pos = jax.lax.broadcasted_iota(jnp.int32, (N, P), 1)
    yc = ((pos // Wf).astype(f32) + 0.5) * (1.0 / Hf)
    xc = ((pos % Wf).astype(f32) + 0.5) * (1.0 / Wf)
    mask = ((yc >= y1) & (yc <= y2) & (xc >= x1) & (xc <= x2)).astype(f32)
    denom = jnp.maximum(jnp.sum(mask, axis=1, keepdims=True), 1.0)
    onehot = (obj_ref[...] == b).astype(f32)
    mb = (mask * onehot).astype(bf16)

    frame_b = frame_ref[0].astype(bf16)

    pooled = jax.lax.dot_general(mb, frame_b, (((1,), (1,)), ((), ())),
                                 preferred_element_type=f32) / denom
    q = jnp.maximum(
        jnp.dot(pooled.astype(bf16), wobj_bf_ref[...],
                preferred_element_type=f32)
        + bobj_ref[...], 0.0)
    keym = jnp.maximum(
        jax.lax.dot_general(frame_b, wkey_bf_ref[...],
                            (((0,), (0,)), ((), ())),
                            preferred_element_type=f32) + bkey_ref[...], 0.0)
    valm = jnp.maximum(
        jax.lax.dot_general(frame_b, wval_bf_ref[...],
                            (((0,), (0,)), ((), ())),
                            preferred_element_type=f32) + bval_ref[...], 0.0)
    scores = jax.lax.dot_general(q, keym, (((1,), (1,)), ((), ())),
                                 preferred_element_type=f32)
    m = jnp.max(scores, axis=1, keepdims=True)
    e = jnp.exp(scores - m)
    attn = (e / jnp.sum(e, axis=1, keepdims=True)) * onehot
    att = jnp.dot(attn, valm, preferred_element_type=f32)

    @pl.when(b == 0)
    def _():
        out_ref[:, :C] = pooled
        att_acc_ref[...] = att

    @pl.when(b != 0)
    def _():
        out_ref[:, :C] += pooled
        att_acc_ref[...] += att

    @pl.when(b == nb - 1)
    def _():
        ctx = jnp.maximum(
            jnp.dot(att_acc_ref[...].astype(bf16), wctx_bf_ref[...],
                    preferred_element_type=f32) + bctx_ref[...], 0.0)
        out_ref[:, C:] = ctx


@jax.jit
def kernel(frame_deep_features, bboxes, obj_slicing, W_obj, b_obj, W_key,
           b_key, W_val, b_val, W_ctx, b_ctx):
    B, C, Hf, Wf = frame_deep_features.shape
    N = bboxes.shape[0]
    P = Hf * Wf
    Dq = W_obj.shape[1]
    Dc = W_ctx.shape[1]
    frame_flat = frame_deep_features.reshape(B, C, P)
    obj2 = obj_slicing.reshape(N, 1)

    return pl.pallas_call(
        functools.partial(_vb_kernel, Hf, Wf),
        grid=(B,),
        in_specs=[
            pl.BlockSpec((N, 4), lambda b: (0, 0)),
            pl.BlockSpec((N, 1), lambda b: (0, 0)),
            pl.BlockSpec((1, C, P), lambda b: (b, 0, 0)),
            pl.BlockSpec((C, Dq), lambda b: (0, 0)),
            pl.BlockSpec((1, Dq), lambda b: (0, 0)),
            pl.BlockSpec((C, Dq), lambda b: (0, 0)),
            pl.BlockSpec((1, Dq), lambda b: (0, 0)),
            pl.BlockSpec((C, Dq), lambda b: (0, 0)),
            pl.BlockSpec((1, Dq), lambda b: (0, 0)),
            pl.BlockSpec((Dq, Dc), lambda b: (0, 0)),
            pl.BlockSpec((1, Dc), lambda b: (0, 0)),
        ],
        out_specs=pl.BlockSpec((N, C + Dc), lambda b: (0, 0)),
        out_shape=jax.ShapeDtypeStruct((N, C + Dc), jnp.float32),
        scratch_shapes=[pltpu.VMEM((N, Dq), jnp.float32),
                        pltpu.VMEM((C, Dq), jnp.bfloat16),
                        pltpu.VMEM((C, Dq), jnp.bfloat16),
                        pltpu.VMEM((C, Dq), jnp.bfloat16),
                        pltpu.VMEM((Dq, Dc), jnp.bfloat16)],
    )(bboxes, obj2, frame_flat, W_obj, b_obj.reshape(1, Dq),
      W_key, b_key.reshape(1, Dq), W_val,
      b_val.reshape(1, Dq), W_ctx, b_ctx.reshape(1, Dc))


# speedup vs baseline: 1.7233x; 1.0718x over previous
import functools

import jax
import jax.numpy as jnp
from jax.experimental import pallas as pl
from jax.experimental.pallas import tpu as pltpu


def _vb_kernel(Hf, Wf, bbox_ref, obj_ref, frame_ref, wobj_hbm, bobj_ref,
               wkey_hbm, bkey_ref, wval_hbm, bval_ref, wctx_hbm, bctx_ref,
               out_ref, wkey_v, wval_v, wobj_v, wctx_v, kv_ref, sems):
    b = pl.program_id(0)
    nb = pl.num_programs(0)
    f32 = jnp.float32
    bf16 = jnp.bfloat16
    N = bbox_ref.shape[0]
    C, P = frame_ref.shape[1], frame_ref.shape[2]
    Dq = wobj_v.shape[1]

    cp_key = pltpu.make_async_copy(wkey_hbm, wkey_v, sems.at[0])
    cp_val = pltpu.make_async_copy(wval_hbm, wval_v, sems.at[1])
    cp_obj = pltpu.make_async_copy(wobj_hbm, wobj_v, sems.at[2])
    cp_ctx = pltpu.make_async_copy(wctx_hbm, wctx_v, sems.at[3])

    @pl.when(b == 0)
    def _():
        cp_key.start()
        cp_val.start()
        cp_obj.start()
        cp_ctx.start()

    bx = bbox_ref[...]
    x1 = jnp.minimum(bx[:, 0:1], bx[:, 2:3])
    x2 = jnp.maximum(bx[:, 0:1], bx[:, 2:3])
    y1 = jnp.minimum(bx[:, 1:2], bx[:, 3:4])
    y2 = jnp.maximum(bx[:, 1:2], bx[:, 3:4])
    pos = jax.lax.broadcasted_iota(jnp.int32, (N, P), 1)
    yc = ((pos // Wf).astype(f32) + 0.5) * (1.0 / Hf)
    xc = ((pos % Wf).astype(f32) + 0.5) * (1.0 / Wf)
    mask = ((yc >= y1) & (yc <= y2) & (xc >= x1) & (xc <= x2)).astype(f32)
    denom = jnp.maximum(jnp.sum(mask, axis=1, keepdims=True), 1.0)
    onehot = (obj_ref[...] == b).astype(f32)
    mb = (mask * onehot).astype(bf16)

    frame_b = frame_ref[0].astype(bf16)

    pooled = jax.lax.dot_general(mb, frame_b, (((1,), (1,)), ((), ())),
                                 preferred_element_type=f32) / denom

    @pl.when(b == 0)
    def _():
        out_ref[:, :C] = pooled
        cp_key.wait()
        cp_val.wait()

    @pl.when(b != 0)
    def _():
        out_ref[:, :C] += pooled

    keym = jnp.maximum(
        jax.lax.dot_general(frame_b, wkey_v[...].astype(bf16),
                            (((0,), (0,)), ((), ())),
                            preferred_element_type=f32) + bkey_ref[...], 0.0)
    valm = jnp.maximum(
        jax.lax.dot_general(frame_b, wval_v[...].astype(bf16),
                            (((0,), (0,)), ((), ())),
                            preferred_element_type=f32) + bval_ref[...], 0.0)
    kv_ref[b, :, :Dq] = keym.astype(bf16)
    kv_ref[b, :, Dq:] = valm.astype(bf16)

    @pl.when(b == nb - 1)
    def _():
        cp_obj.wait()
        q = jnp.maximum(
            jnp.dot(out_ref[:, :C].astype(bf16), wobj_v[...].astype(bf16),
                    preferred_element_type=f32) + bobj_ref[...], 0.0)
        qb = q.astype(bf16)
        att = jnp.zeros((N, Dq), dtype=f32)
        for bb in range(nb):
            kvb = kv_ref[bb]
            scores = jax.lax.dot_general(
                qb, kvb[:, :Dq], (((1,), (1,)), ((), ())),
                preferred_element_type=f32)
            m = jnp.max(scores, axis=1, keepdims=True)
            e = jnp.exp(scores - m)
            attn = ((e / jnp.sum(e, axis=1, keepdims=True))
                    * (obj_ref[...] == bb).astype(f32))
            att = att + jnp.dot(attn.astype(bf16), kvb[:, Dq:],
                                preferred_element_type=f32)
        cp_ctx.wait()
        ctx = jnp.maximum(
            jnp.dot(att.astype(bf16), wctx_v[...].astype(bf16),
                    preferred_element_type=f32) + bctx_ref[...], 0.0)
        out_ref[:, C:] = ctx


@jax.jit
def kernel(frame_deep_features, bboxes, obj_slicing, W_obj, b_obj, W_key,
           b_key, W_val, b_val, W_ctx, b_ctx):
    B, C, Hf, Wf = frame_deep_features.shape
    N = bboxes.shape[0]
    P = Hf * Wf
    Dq = W_obj.shape[1]
    Dc = W_ctx.shape[1]
    frame_flat = frame_deep_features.reshape(B, C, P)
    obj2 = obj_slicing.reshape(N, 1)
    anyspec = pl.BlockSpec(memory_space=pl.ANY)

    return pl.pallas_call(
        functools.partial(_vb_kernel, Hf, Wf),
        grid=(B,),
        in_specs=[
            pl.BlockSpec((N, 4), lambda b: (0, 0)),
            pl.BlockSpec((N, 1), lambda b: (0, 0)),
            pl.BlockSpec((1, C, P), lambda b: (b, 0, 0)),
            anyspec,
            pl.BlockSpec((1, Dq), lambda b: (0, 0)),
            anyspec,
            pl.BlockSpec((1, Dq), lambda b: (0, 0)),
            anyspec,
            pl.BlockSpec((1, Dq), lambda b: (0, 0)),
            anyspec,
            pl.BlockSpec((1, Dc), lambda b: (0, 0)),
        ],
        out_specs=pl.BlockSpec((N, C + Dc), lambda b: (0, 0)),
        out_shape=jax.ShapeDtypeStruct((N, C + Dc), jnp.float32),
        scratch_shapes=[
            pltpu.VMEM((C, Dq), jnp.float32),
            pltpu.VMEM((C, Dq), jnp.float32),
            pltpu.VMEM((C, Dq), jnp.float32),
            pltpu.VMEM((Dq, Dc), jnp.float32),
            pltpu.VMEM((B, P, 2 * Dq), jnp.bfloat16),
            pltpu.SemaphoreType.DMA((4,)),
        ],
    )(bboxes, obj2, frame_flat, W_obj, b_obj.reshape(1, Dq),
      W_key, b_key.reshape(1, Dq), W_val,
      b_val.reshape(1, Dq), W_ctx, b_ctx.reshape(1, Dc))
